# Initial kernel scaffold; baseline (speedup 1.0000x reference)
import jax
import jax.numpy as jnp
from jax.experimental import pallas as pl


def kernel(x_user, x_item, x_taste, x_intention, x_image, nutrient, ei_taste_ing, ei_taste_item, ei_int_item, ei_img_item, ei_user_item, ei_item_user, params):
    raise NotImplementedError("write your pallas kernel here")


# speedup vs baseline: 1.0010x; 1.0010x over previous
import functools

import jax
import jax.numpy as jnp
from jax import lax
from jax.experimental import pallas as pl
from jax.experimental.pallas import tpu as pltpu

HID = 128
EDGE_TYPES = [('taste', 'associated_with', 'item'),
              ('intention', 'associated_with', 'item'),
              ('image', 'associated_with', 'item'),
              ('user', 'buys', 'item'),
              ('item', 'bought_by', 'user')]


def _ename(et):
    return et[0] + '__' + et[1] + '__' + et[2]


def _bn(x, g, b, eps=1e-5):
    m = jnp.mean(x, axis=0)
    v = jnp.var(x, axis=0)
    return (x - m) / jnp.sqrt(v + eps) * g + b


def _encoder(x, p):
    h = jax.nn.relu(x @ p['ce_W1'] + p['ce_b1'])
    z = h @ p['ce_W2'] + p['ce_b2']
    nrm = jnp.sqrt(jnp.sum(z * z, axis=1, keepdims=True))
    return z / jnp.maximum(nrm, 1e-12)


_CL_BLK = 512


def _cl_body(z1_ref, z2_ref, acc_ref):
    i = pl.program_id(0)
    sim = jnp.dot(z1_ref[...], z2_ref[...].T,
                  preferred_element_type=jnp.float32) * 2.0
    mx = jnp.max(sim, axis=1, keepdims=True)
    lse = jnp.log(jnp.sum(jnp.exp(sim - mx), axis=1)) + mx[:, 0]
    rows = i * _CL_BLK + lax.broadcasted_iota(jnp.int32, (_CL_BLK, 1), 0)
    cols = lax.broadcasted_iota(jnp.int32, (_CL_BLK, sim.shape[1]), 1)
    diag = jnp.sum(jnp.where(cols == rows, sim, 0.0), axis=1)
    part = jnp.sum(lse - diag)

    @pl.when(i == 0)
    def _():
        acc_ref[0, 0] = 0.0

    acc_ref[0, 0] += part


def _cl_loss(z1, z2):
    n = z1.shape[0]
    acc = pl.pallas_call(
        _cl_body,
        grid=(n // _CL_BLK,),
        in_specs=[pl.BlockSpec((_CL_BLK, HID), lambda i: (i, 0)),
                  pl.BlockSpec((n, HID), lambda i: (0, 0))],
        out_specs=pl.BlockSpec((1, 1), lambda i: (0, 0), memory_space=pltpu.SMEM),
        out_shape=jax.ShapeDtypeStruct((1, 1), jnp.float32),
    )(z1, z2)
    return acc[0, 0] / n


def _lgconv(x, edge_index):
    src, dst = edge_index[0], edge_index[1]
    n = x.shape[0]
    deg = jnp.zeros((n,), x.dtype).at[dst].add(1.0)
    dinv = jnp.where(deg > 0, 1.0 / jnp.sqrt(jnp.maximum(deg, 1e-12)), 0.0)
    norm = dinv[src] * dinv[dst]
    return jnp.zeros_like(x).at[dst].add(norm[:, None] * x[src])


def _hgt(x_dict, eidx, p):
    D = float(HID)
    k = {t: x_dict[t] @ p['Wk_' + t] + p['bk_' + t] for t in x_dict}
    q = {t: x_dict[t] @ p['Wq_' + t] + p['bq_' + t] for t in x_dict}
    v = {t: x_dict[t] @ p['Wv_' + t] + p['bv_' + t] for t in x_dict}
    per_dst = {}
    for et in EDGE_TYPES:
        s, r, d = et
        name = _ename(et)
        e = eidx[name]
        src, dst = e[0], e[1]
        k_rel = k[s] @ p['Ak_' + name]
        v_rel = v[s] @ p['Av_' + name]
        logit = jnp.sum(q[d][dst] * k_rel[src], axis=-1) * p['prel_' + name] / jnp.sqrt(D)
        per_dst.setdefault(d, []).append((logit, v_rel[src], dst))
    out = {}
    for d, items in per_dst.items():
        logits = jnp.concatenate([a for a, _, _ in items], axis=0)
        msgs = jnp.concatenate([m for _, m, _ in items], axis=0)
        dsts = jnp.concatenate([i for _, _, i in items], axis=0)
        nd = x_dict[d].shape[0]
        mx = jax.ops.segment_max(logits, dsts, num_segments=nd)
        mx = jnp.where(jnp.isfinite(mx), mx, 0.0)
        ex = jnp.exp(logits - mx[dsts])
        den = jax.ops.segment_sum(ex, dsts, num_segments=nd)
        alpha = ex / jnp.maximum(den[dsts], 1e-16)
        agg = jax.ops.segment_sum(alpha[:, None] * msgs, dsts, num_segments=nd)
        a = jax.nn.gelu(agg, approximate=False) @ p['Wa_' + d] + p['ba_' + d]
        beta = jax.nn.sigmoid(p['skip_' + d])
        out[d] = beta * a + (1.0 - beta) * x_dict[d]
    return out


def kernel(x_user, x_item, x_taste, x_intention, x_image, nutrient,
           ei_taste_ing, ei_taste_item, ei_int_item, ei_img_item,
           ei_user_item, ei_item_user, params):
    p = params
    nut = _bn(jax.nn.relu(nutrient @ p['Wnp'] + p['bnp']), p['nbn_g'], p['nbn_b'])
    z1 = _encoder(nut, p)
    z2 = _encoder(x_intention, p)
    cl_loss = _cl_loss(z1, z2)
    x = {'user': _bn(x_user, p['bn_user_g'], p['bn_user_b']),
         'item': _bn(x_item, p['bn_item_g'], p['bn_item_b']),
         'intention': _bn(x_intention, p['bn_int_g'], p['bn_int_b']),
         'image': _bn(x_image, p['bn_img_g'], p['bn_img_b']),
         'taste': _bn(x_taste, p['bn_taste_g'], p['bn_taste_b'])}
    x['taste'] = _lgconv(x['taste'], ei_taste_ing)
    eidx = {'taste__associated_with__item': ei_taste_item,
            'intention__associated_with__item': ei_int_item,
            'image__associated_with__item': ei_img_item,
            'user__buys__item': ei_user_item,
            'item__bought_by__user': ei_item_user}
    fused = _hgt(x, eidx, p)
    x.update(fused)
    return (x['user'], x['item'], x['taste'], x['intention'], x['image'], cl_loss)


# speedup vs baseline: 3.8847x; 3.8808x over previous
import functools

import jax
import jax.numpy as jnp
from jax import lax
from jax.experimental import pallas as pl
from jax.experimental.pallas import tpu as pltpu
from jax.experimental.pallas import tpu_sc as plsc

HID = 128
NC, NS, NW = 2, 16, 32
BA = 128
BB = 128

EDGE_TYPES = [('taste', 'associated_with', 'item'),
              ('intention', 'associated_with', 'item'),
              ('image', 'associated_with', 'item'),
              ('user', 'buys', 'item'),
              ('item', 'bought_by', 'user')]


def _ename(et):
    return et[0] + '__' + et[1] + '__' + et[2]


def _bn(x, g, b, eps=1e-5):
    m = jnp.mean(x, axis=0)
    v = jnp.var(x, axis=0)
    return (x - m) / jnp.sqrt(v + eps) * g + b


def _encoder(x, p):
    h = jax.nn.relu(x @ p['ce_W1'] + p['ce_b1'])
    z = h @ p['ce_W2'] + p['ce_b2']
    nrm = jnp.sqrt(jnp.sum(z * z, axis=1, keepdims=True))
    return z / jnp.maximum(nrm, 1e-12)


def _mesh():
    return plsc.VectorSubcoreMesh(core_axis_name="c", subcore_axis_name="s",
                                  num_cores=NC, num_subcores=NS)


_SC_PARAMS = pltpu.CompilerParams(needs_layout_passes=False,
                                  use_tc_tiling_on_sc=False)


def _mk_phase_a(E_pad, ND_pad):
    Esh = E_pad // NW
    NB = Esh // BA
    assert Esh % BA == 0 and ND_pad % 16 == 0

    @functools.partial(
        pl.kernel,
        out_type=(jax.ShapeDtypeStruct((E_pad,), jnp.float32),
                  jax.ShapeDtypeStruct((NW, ND_pad), jnp.float32)),
        mesh=_mesh(),
        compiler_params=_SC_PARAMS,
        scratch_types=[pltpu.VMEM((1, BA), jnp.int32),
                       pltpu.VMEM((1, BA), jnp.int32),
                       pltpu.VMEM((BA,), jnp.float32),
                       pltpu.VMEM((BA, HID), jnp.float32),
                       pltpu.VMEM((BA, HID), jnp.float32),
                       pltpu.VMEM((ND_pad,), jnp.float32),
                       pltpu.SemaphoreType.DMA,
                       pltpu.SemaphoreType.DMA],
    )
    def kern(srcg, dst, ktab, qtab, ex_out, den_out,
             src_i, dst_i, ex_b, krows, qrows, den_v, sem1, sem2):
        c = lax.axis_index("c")
        s = lax.axis_index("s")
        wid = s * NC + c
        base_t = wid * Esh
        iota16 = lax.iota(jnp.int32, 16)
        zeros16 = jnp.zeros((16,), jnp.float32)

        def zden(i, carry):
            plsc.store_scatter(den_v, [i * 16 + iota16], zeros16)
            return carry
        lax.fori_loop(0, ND_pad // 16, zden, 0)

        def blk(bi, carry):
            base = base_t + bi * BA
            pltpu.sync_copy(srcg.at[pl.ds(base, BA)], src_i.at[0])
            pltpu.sync_copy(dst.at[pl.ds(base, BA)], dst_i.at[0])
            cp1 = pltpu.async_copy(ktab.at[src_i.at[0]], krows, sem1)
            cp2 = pltpu.async_copy(qtab.at[dst_i.at[0]], qrows, sem2)
            cp1.wait()
            cp2.wait()
            for g in range(BA // 16):
                rows16 = g * 16 + iota16

                def dot8(i, acc):
                    for cc in range(8):
                        fv = jnp.full((16,), i * 8 + cc, jnp.int32)
                        kc = plsc.load_gather(krows, [rows16, fv])
                        qc = plsc.load_gather(qrows, [rows16, fv])
                        acc = acc + kc * qc
                    return acc
                acc = lax.fori_loop(0, HID // 8, dot8,
                                    jnp.zeros((16,), jnp.float32))
                ex_vec = jnp.exp(acc)
                ex_b[pl.ds(g * 16, 16)] = ex_vec
                dvec = dst_i[0, pl.ds(g * 16, 16)]
                plsc.addupdate_scatter(den_v, [dvec], ex_vec)
            pltpu.sync_copy(ex_b, ex_out.at[pl.ds(base, BA)])
            return carry
        lax.fori_loop(0, NB, blk, 0)
        pltpu.sync_copy(den_v, den_out.at[wid])

    return kern


def _mk_phase_b(E_pad, ND_pad, lg, DT_pad=0):
    Esh = E_pad // NS
    NB = Esh // BB
    r_t = ND_pad // NS
    assert Esh % BB == 0 and ND_pad % (NS * 64) == 0

    scratch = [pltpu.VMEM((1, BB), jnp.int32),
               pltpu.VMEM((1, BB), jnp.int32),
               pltpu.VMEM((1, BB), jnp.float32),
               pltpu.VMEM((BB, 32), jnp.float32),
               pltpu.VMEM((64, 32), jnp.float32),
               pltpu.VMEM_SHARED((ND_pad, 32), jnp.float32),
               pltpu.SemaphoreType.DMA]
    if lg:
        scratch.append(pltpu.VMEM((DT_pad,), jnp.float32))

    @functools.partial(
        pl.kernel,
        out_type=jax.ShapeDtypeStruct((4, ND_pad, 32), jnp.float32),
        mesh=_mesh(),
        compiler_params=_SC_PARAMS,
        scratch_types=scratch,
    )
    def kern(srcg, dst, warr, v0, v1, v2, v3, out, *scr):
        if lg:
            src_i, dst_i, w_b, vrows, zbuf, agg, sem, dinv_v = scr
        else:
            src_i, dst_i, w_b, vrows, zbuf, agg, sem = scr
        vs = (v0, v1, v2, v3)
        c = lax.axis_index("c")
        s = lax.axis_index("s")
        iota16 = lax.iota(jnp.int32, 16)
        z16i = jnp.zeros((16,), jnp.int32)
        zeros16 = jnp.zeros((16,), jnp.float32)
        for r in range(64):
            zbuf[r, pl.ds(0, 16)] = zeros16
            zbuf[r, pl.ds(16, 16)] = zeros16
        if lg:
            pltpu.sync_copy(warr, dinv_v)

        for chunk in range(4):
            @pl.when(c == chunk // 2)
            def _(chunk=chunk):
                def zrow(i, carry):
                    pltpu.sync_copy(zbuf, agg.at[pl.ds(s * r_t + i * 64, 64)])
                    return carry
                lax.fori_loop(0, r_t // 64, zrow, 0)
                plsc.subcore_barrier()

                def blk(bi, carry):
                    base = s * Esh + bi * BB
                    pltpu.sync_copy(srcg.at[pl.ds(base, BB)], src_i.at[0])
                    pltpu.sync_copy(dst.at[pl.ds(base, BB)], dst_i.at[0])
                    if not lg:
                        pltpu.sync_copy(warr.at[pl.ds(base, BB)], w_b.at[0])
                    cp = pltpu.async_copy(vs[chunk].at[src_i.at[0]], vrows, sem)
                    cp.wait()
                    if lg:
                        for g in range(BB // 16):
                            svec = src_i[0, pl.ds(g * 16, 16)]
                            dvec = dst_i[0, pl.ds(g * 16, 16)]
                            wv = (plsc.load_gather(dinv_v, [svec]) *
                                  plsc.load_gather(dinv_v, [dvec]))
                            w_b[0, pl.ds(g * 16, 16)] = wv

                    def sca(e, carry):
                        ev = jnp.full((16,), e, jnp.int32)
                        w16 = plsc.load_gather(w_b, [z16i, ev])
                        lo = plsc.load_gather(vrows, [ev, iota16])
                        hi = plsc.load_gather(vrows, [ev, iota16 + 16])
                        plsc.store_scatter(vrows, [ev, iota16], lo * w16)
                        plsc.store_scatter(vrows, [ev, iota16 + 16], hi * w16)
                        return carry
                    lax.fori_loop(0, BB, sca, 0)
                    pltpu.sync_copy(vrows, agg.at[dst_i.at[0]], add=True)
                    return carry
                lax.fori_loop(0, NB, blk, 0)
                plsc.subcore_barrier()
                pltpu.sync_copy(agg.at[pl.ds(s * r_t, r_t)],
                                out.at[chunk, pl.ds(s * r_t, r_t)])
                plsc.subcore_barrier()

    return kern


def _mk_deg(E_pad, ND_pad):
    Esh = E_pad // NW
    NB = Esh // BA

    @functools.partial(
        pl.kernel,
        out_type=jax.ShapeDtypeStruct((NW, ND_pad), jnp.float32),
        mesh=_mesh(),
        compiler_params=_SC_PARAMS,
        scratch_types=[pltpu.VMEM((1, BA), jnp.int32),
                       pltpu.VMEM((ND_pad,), jnp.float32)],
    )
    def kern(dst, deg_out, dst_i, deg_v):
        c = lax.axis_index("c")
        s = lax.axis_index("s")
        wid = s * NC + c
        base_t = wid * Esh
        iota16 = lax.iota(jnp.int32, 16)
        zeros16 = jnp.zeros((16,), jnp.float32)
        ones16 = jnp.ones((16,), jnp.float32)

        def zden(i, carry):
            plsc.store_scatter(deg_v, [i * 16 + iota16], zeros16)
            return carry
        lax.fori_loop(0, ND_pad // 16, zden, 0)

        def blk(bi, carry):
            pltpu.sync_copy(dst.at[pl.ds(base_t + bi * BA, BA)], dst_i.at[0])
            for g in range(BA // 16):
                dvec = dst_i[0, pl.ds(g * 16, 16)]
                plsc.addupdate_scatter(deg_v, [dvec], ones16)
            return carry
        lax.fori_loop(0, NB, blk, 0)
        pltpu.sync_copy(deg_v, deg_out.at[wid])

    return kern


_CL_BLK = 512


def _cl_body(z1_ref, z2_ref, acc_ref):
    i = pl.program_id(0)
    sim = jnp.dot(z1_ref[...], z2_ref[...].T,
                  preferred_element_type=jnp.float32) * 2.0
    mx = jnp.max(sim, axis=1, keepdims=True)
    lse = jnp.log(jnp.sum(jnp.exp(sim - mx), axis=1)) + mx[:, 0]
    rows = i * _CL_BLK + lax.broadcasted_iota(jnp.int32, (_CL_BLK, 1), 0)
    cols = lax.broadcasted_iota(jnp.int32, (_CL_BLK, sim.shape[1]), 1)
    diag = jnp.sum(jnp.where(cols == rows, sim, 0.0), axis=1)
    part = jnp.sum(lse - diag)

    @pl.when(i == 0)
    def _():
        acc_ref[0, 0] = 0.0

    acc_ref[0, 0] += part


def _cl_loss(z1, z2):
    n = z1.shape[0]
    acc = pl.pallas_call(
        _cl_body,
        grid=(n // _CL_BLK,),
        in_specs=[pl.BlockSpec((_CL_BLK, HID), lambda i: (i, 0)),
                  pl.BlockSpec((n, HID), lambda i: (0, 0))],
        out_specs=pl.BlockSpec((1, 1), lambda i: (0, 0), memory_space=pltpu.SMEM),
        out_shape=jax.ShapeDtypeStruct((1, 1), jnp.float32),
    )(z1, z2)
    return acc[0, 0] / n


def _pad_edges(src, dst, e_pad, dummy):
    e = src.shape[0]
    src_p = jnp.concatenate([src, jnp.zeros((e_pad - e,), jnp.int32)])
    dst_p = jnp.concatenate([dst, jnp.full((e_pad - e,), dummy, jnp.int32)])
    return src_p, dst_p


def _assemble(out_b, nd):
    return out_b.transpose(1, 0, 2).reshape(out_b.shape[1], HID)[:nd]


def kernel(x_user, x_item, x_taste, x_intention, x_image, nutrient,
           ei_taste_ing, ei_taste_item, ei_int_item, ei_img_item,
           ei_user_item, ei_item_user, params):
    p = params
    D = float(HID)

    nut = _bn(jax.nn.relu(nutrient @ p['Wnp'] + p['bnp']), p['nbn_g'], p['nbn_b'])
    z1 = _encoder(nut, p)
    z2 = _encoder(x_intention, p)
    cl_loss = _cl_loss(z1, z2)

    xb = {'user': _bn(x_user, p['bn_user_g'], p['bn_user_b']),
          'item': _bn(x_item, p['bn_item_g'], p['bn_item_b']),
          'intention': _bn(x_intention, p['bn_int_g'], p['bn_int_b']),
          'image': _bn(x_image, p['bn_img_g'], p['bn_img_b']),
          'taste': _bn(x_taste, p['bn_taste_g'], p['bn_taste_b'])}

    NT = xb['taste'].shape[0]
    NDP_T = 32768
    E_LG = ei_taste_ing.shape[1]
    EP_LG = 483328
    lsrc, ldst = _pad_edges(ei_taste_ing[0], ei_taste_ing[1], EP_LG, NT)
    deg_part = _mk_deg(EP_LG, NDP_T)(ldst)
    deg = deg_part.sum(axis=0)[:NT]
    dinv = jnp.where(deg > 0, 1.0 / jnp.sqrt(jnp.maximum(deg, 1e-12)), 0.0)
    dinv_pad = jnp.concatenate([dinv, jnp.zeros((NDP_T - NT,), jnp.float32)])
    xt = xb['taste']
    xt_pad = jnp.concatenate(
        [xt, jnp.zeros((NDP_T - NT, HID), jnp.float32)], axis=0)
    tv = [xt_pad[:, 32 * j:32 * (j + 1)] for j in range(4)]
    lg_b = _mk_phase_b(EP_LG, NDP_T, lg=True, DT_pad=NDP_T)(
        lsrc, ldst, dinv_pad, *tv)
    taste_out = _assemble(lg_b, NT)

    xh = dict(xb)
    xh['taste'] = taste_out
    k_rel, v_rel = {}, {}
    for et in EDGE_TYPES:
        s_t, _, _ = et
        name = _ename(et)
        cc = p['prel_' + name] / jnp.sqrt(D)
        kk = (xh[s_t] @ p['Wk_' + s_t] + p['bk_' + s_t]) @ p['Ak_' + name]
        k_rel[name] = kk * cc
        v_rel[name] = (xh[s_t] @ p['Wv_' + s_t] + p['bv_' + s_t]) @ p['Av_' + name]
    q_item = xh['item'] @ p['Wq_item'] + p['bq_item']
    q_user = xh['user'] @ p['Wq_user'] + p['bq_user']

    rel_item = ['taste__associated_with__item', 'intention__associated_with__item',
                'image__associated_with__item', 'user__buys__item']
    ei_item = [ei_taste_item, ei_int_item, ei_img_item, ei_user_item]
    K_item = jnp.concatenate([k_rel[r] for r in rel_item], axis=0)
    V_item = jnp.concatenate([v_rel[r] for r in rel_item], axis=0)
    offs = [0]
    for r in rel_item[:-1]:
        offs.append(offs[-1] + k_rel[r].shape[0])
    src_item = jnp.concatenate(
        [e[0] + o for e, o in zip(ei_item, offs)])
    dst_item = jnp.concatenate([e[1] for e in ei_item])

    def _hgt_dst(src_g, dst_g, Ktab, Vtab, qtab, e_pad, nd, nd_pad, wa, ba,
                 skip, x_prev):
        src_p, dst_p = _pad_edges(src_g, dst_g, e_pad, nd)
        q_pad = jnp.concatenate(
            [qtab, jnp.zeros((nd_pad - nd, HID), jnp.float32)], axis=0)
        ex, den_part = _mk_phase_a(e_pad, nd_pad)(src_p, dst_p, Ktab, q_pad)
        vv = [Vtab[:, 32 * j:32 * (j + 1)] for j in range(4)]
        agg_b = _mk_phase_b(e_pad, nd_pad, lg=False)(src_p, dst_p, ex, *vv)
        agg = _assemble(agg_b, nd)
        den = den_part.sum(axis=0)[:nd]
        aggn = agg / jnp.maximum(den, 1e-16)[:, None]
        a = jax.nn.gelu(aggn, approximate=False) @ wa + ba
        beta = jax.nn.sigmoid(skip)
        return beta * a + (1.0 - beta) * x_prev

    item_out = _hgt_dst(src_item, dst_item, K_item, V_item, q_item,
                        1720320, 50000, 53248, p['Wa_item'], p['ba_item'],
                        p['skip_item'], xh['item'])
    user_out = _hgt_dst(ei_item_user[0], ei_item_user[1],
                        k_rel['item__bought_by__user'],
                        v_rel['item__bought_by__user'], q_user,
                        643072, 20000, 20480, p['Wa_user'], p['ba_user'],
                        p['skip_user'], xh['user'])

    return (user_out, item_out, taste_out, xb['intention'], xb['image'], cl_loss)


# speedup vs baseline: 6.5667x; 1.6904x over previous
import functools

import jax
import jax.numpy as jnp
from jax import lax
from jax.experimental import pallas as pl
from jax.experimental.pallas import tpu as pltpu
from jax.experimental.pallas import tpu_sc as plsc

HID = 128
NC, NS, NW = 2, 16, 32
BA = 128
BB = 128

EDGE_TYPES = [('taste', 'associated_with', 'item'),
              ('intention', 'associated_with', 'item'),
              ('image', 'associated_with', 'item'),
              ('user', 'buys', 'item'),
              ('item', 'bought_by', 'user')]


def _ename(et):
    return et[0] + '__' + et[1] + '__' + et[2]


def _bn(x, g, b, eps=1e-5):
    m = jnp.mean(x, axis=0)
    v = jnp.var(x, axis=0)
    return (x - m) / jnp.sqrt(v + eps) * g + b


def _encoder(x, p):
    h = jax.nn.relu(x @ p['ce_W1'] + p['ce_b1'])
    z = h @ p['ce_W2'] + p['ce_b2']
    nrm = jnp.sqrt(jnp.sum(z * z, axis=1, keepdims=True))
    return z / jnp.maximum(nrm, 1e-12)


def _mesh():
    return plsc.VectorSubcoreMesh(core_axis_name="c", subcore_axis_name="s",
                                  num_cores=NC, num_subcores=NS)


_SC_PARAMS = pltpu.CompilerParams(needs_layout_passes=False,
                                  use_tc_tiling_on_sc=False)


def _mk_phase_a(E_pad, ND_pad):
    Esh = E_pad // NW
    NB = Esh // BA
    assert Esh % BA == 0 and NB % 2 == 0 and ND_pad % 16 == 0

    @functools.partial(
        pl.kernel,
        out_type=(jax.ShapeDtypeStruct((E_pad,), jnp.float32),
                  jax.ShapeDtypeStruct((NW, ND_pad), jnp.float32)),
        mesh=_mesh(),
        compiler_params=_SC_PARAMS,
        scratch_types=[pltpu.VMEM((1, BA), jnp.int32),
                       pltpu.VMEM((1, BA), jnp.int32),
                       pltpu.VMEM((1, BA), jnp.int32),
                       pltpu.VMEM((1, BA), jnp.int32),
                       pltpu.VMEM((BA,), jnp.float32),
                       pltpu.VMEM((BA, HID), jnp.float32),
                       pltpu.VMEM((BA, HID), jnp.float32),
                       pltpu.VMEM((BA, HID), jnp.float32),
                       pltpu.VMEM((BA, HID), jnp.float32),
                       pltpu.VMEM((16, 137), jnp.float32),
                       pltpu.VMEM((ND_pad,), jnp.float32),
                       pltpu.SemaphoreType.DMA,
                       pltpu.SemaphoreType.DMA,
                       pltpu.SemaphoreType.DMA,
                       pltpu.SemaphoreType.DMA],
    )
    def kern(srcg, dst, ktab, qtab, ex_out, den_out,
             src_i0, dst_i0, src_i1, dst_i1, ex_b,
             krows0, qrows0, krows1, qrows1, acc_t, den_v,
             semk0, semq0, semk1, semq1):
        c = lax.axis_index("c")
        s = lax.axis_index("s")
        wid = s * NC + c
        base_t = wid * Esh
        iota16 = lax.iota(jnp.int32, 16)
        zeros16 = jnp.zeros((16,), jnp.float32)
        src_i = (src_i0, src_i1)
        dst_i = (dst_i0, dst_i1)
        krows = (krows0, krows1)
        qrows = (qrows0, qrows1)
        semk = (semk0, semk1)
        semq = (semq0, semq1)

        def zden(i, carry):
            plsc.store_scatter(den_v, [i * 16 + iota16], zeros16)
            return carry
        lax.fori_loop(0, ND_pad // 16, zden, 0)

        def fire(bi, b):
            base = base_t + bi * BA
            pltpu.sync_copy(srcg.at[pl.ds(base, BA)], src_i[b].at[0])
            pltpu.sync_copy(dst.at[pl.ds(base, BA)], dst_i[b].at[0])
            pltpu.async_copy(ktab.at[src_i[b].at[0]], krows[b], semk[b])
            pltpu.async_copy(qtab.at[dst_i[b].at[0]], qrows[b], semq[b])

        def wait(b):
            pltpu.make_async_copy(ktab.at[pl.ds(0, BA)], krows[b], semk[b]).wait()
            pltpu.make_async_copy(qtab.at[pl.ds(0, BA)], qrows[b], semq[b]).wait()

        def compute(bi, b):
            kr, qr = krows[b], qrows[b]
            for g in range(BA // 16):

                def edot(i, carry):
                    ev = jnp.full((16,), g * 16 + i, jnp.int32)
                    acc = zeros16
                    for cc in range(8):
                        fvec = 16 * cc + iota16
                        kv = plsc.load_gather(kr, [ev, fvec])
                        qv = plsc.load_gather(qr, [ev, fvec])
                        acc = acc + kv * qv
                    plsc.store_scatter(
                        acc_t, [iota16, jnp.full((16,), i, jnp.int32)], acc)
                    return carry
                lax.fori_loop(0, 16, edot, 0)
                sum16 = acc_t[0, pl.ds(0, 16)]
                for f in range(1, 16):
                    sum16 = sum16 + acc_t[f, pl.ds(0, 16)]
                ex_vec = jnp.exp(sum16)
                ex_b[pl.ds(g * 16, 16)] = ex_vec
                dvec = dst_i[b][0, pl.ds(g * 16, 16)]
                plsc.addupdate_scatter(den_v, [dvec], ex_vec)
            pltpu.sync_copy(ex_b, ex_out.at[pl.ds(base_t + bi * BA, BA)])

        fire(0, 0)

        def blk2(i, carry):
            wait(0)

            fire(2 * i + 1, 1)
            compute(2 * i, 0)
            wait(1)

            @pl.when(2 * i + 2 < NB)
            def _():
                fire(2 * i + 2, 0)
            compute(2 * i + 1, 1)
            return carry
        lax.fori_loop(0, NB // 2, blk2, 0)
        pltpu.sync_copy(den_v, den_out.at[wid])

    return kern


def _mk_phase_b(E_pad, ND_pad, lg, DT_pad=0):
    Esh = E_pad // NS
    NB = Esh // BB
    r_t = ND_pad // NS
    assert Esh % BB == 0 and ND_pad % (NS * 64) == 0

    scratch = [pltpu.VMEM((1, BB), jnp.int32),
               pltpu.VMEM((1, BB), jnp.int32),
               pltpu.VMEM((1, BB), jnp.float32),
               pltpu.VMEM((BB, 32), jnp.float32),
               pltpu.VMEM((64, 32), jnp.float32),
               pltpu.VMEM_SHARED((ND_pad, 32), jnp.float32),
               pltpu.SemaphoreType.DMA]
    if lg:
        scratch.append(pltpu.VMEM((DT_pad,), jnp.float32))

    @functools.partial(
        pl.kernel,
        out_type=jax.ShapeDtypeStruct((4, ND_pad, 32), jnp.float32),
        mesh=_mesh(),
        compiler_params=_SC_PARAMS,
        scratch_types=scratch,
    )
    def kern(srcg, dst, warr, v0, v1, v2, v3, out, *scr):
        if lg:
            src_i, dst_i, w_b, vrows, zbuf, agg, sem, dinv_v = scr
        else:
            src_i, dst_i, w_b, vrows, zbuf, agg, sem = scr
        vs = (v0, v1, v2, v3)
        c = lax.axis_index("c")
        s = lax.axis_index("s")
        iota16 = lax.iota(jnp.int32, 16)
        z16i = jnp.zeros((16,), jnp.int32)
        zeros16 = jnp.zeros((16,), jnp.float32)
        for r in range(64):
            zbuf[r, pl.ds(0, 16)] = zeros16
            zbuf[r, pl.ds(16, 16)] = zeros16
        if lg:
            pltpu.sync_copy(warr, dinv_v)

        for chunk in range(4):
            @pl.when(c == chunk // 2)
            def _(chunk=chunk):
                def zrow(i, carry):
                    pltpu.sync_copy(zbuf, agg.at[pl.ds(s * r_t + i * 64, 64)])
                    return carry
                lax.fori_loop(0, r_t // 64, zrow, 0)
                plsc.subcore_barrier()

                def blk(bi, carry):
                    base = s * Esh + bi * BB
                    pltpu.sync_copy(srcg.at[pl.ds(base, BB)], src_i.at[0])
                    pltpu.sync_copy(dst.at[pl.ds(base, BB)], dst_i.at[0])
                    if not lg:
                        pltpu.sync_copy(warr.at[pl.ds(base, BB)], w_b.at[0])
                    cp = pltpu.async_copy(vs[chunk].at[src_i.at[0]], vrows, sem)
                    cp.wait()
                    if lg:
                        for g in range(BB // 16):
                            svec = src_i[0, pl.ds(g * 16, 16)]
                            dvec = dst_i[0, pl.ds(g * 16, 16)]
                            wv = (plsc.load_gather(dinv_v, [svec]) *
                                  plsc.load_gather(dinv_v, [dvec]))
                            w_b[0, pl.ds(g * 16, 16)] = wv

                    def sca(e, carry):
                        ev = jnp.full((16,), e, jnp.int32)
                        w16 = plsc.load_gather(w_b, [z16i, ev])
                        lo = plsc.load_gather(vrows, [ev, iota16])
                        hi = plsc.load_gather(vrows, [ev, iota16 + 16])
                        plsc.store_scatter(vrows, [ev, iota16], lo * w16)
                        plsc.store_scatter(vrows, [ev, iota16 + 16], hi * w16)
                        return carry
                    lax.fori_loop(0, BB, sca, 0)
                    pltpu.sync_copy(vrows, agg.at[dst_i.at[0]], add=True)
                    return carry
                lax.fori_loop(0, NB, blk, 0)
                plsc.subcore_barrier()
                pltpu.sync_copy(agg.at[pl.ds(s * r_t, r_t)],
                                out.at[chunk, pl.ds(s * r_t, r_t)])
                plsc.subcore_barrier()

    return kern


def _mk_deg(E_pad, ND_pad):
    Esh = E_pad // NW
    NB = Esh // BA

    @functools.partial(
        pl.kernel,
        out_type=jax.ShapeDtypeStruct((NW, ND_pad), jnp.float32),
        mesh=_mesh(),
        compiler_params=_SC_PARAMS,
        scratch_types=[pltpu.VMEM((1, BA), jnp.int32),
                       pltpu.VMEM((ND_pad,), jnp.float32)],
    )
    def kern(dst, deg_out, dst_i, deg_v):
        c = lax.axis_index("c")
        s = lax.axis_index("s")
        wid = s * NC + c
        base_t = wid * Esh
        iota16 = lax.iota(jnp.int32, 16)
        zeros16 = jnp.zeros((16,), jnp.float32)
        ones16 = jnp.ones((16,), jnp.float32)

        def zden(i, carry):
            plsc.store_scatter(deg_v, [i * 16 + iota16], zeros16)
            return carry
        lax.fori_loop(0, ND_pad // 16, zden, 0)

        def blk(bi, carry):
            pltpu.sync_copy(dst.at[pl.ds(base_t + bi * BA, BA)], dst_i.at[0])
            for g in range(BA // 16):
                dvec = dst_i[0, pl.ds(g * 16, 16)]
                plsc.addupdate_scatter(deg_v, [dvec], ones16)
            return carry
        lax.fori_loop(0, NB, blk, 0)
        pltpu.sync_copy(deg_v, deg_out.at[wid])

    return kern


_CL_BLK = 512


def _cl_body(z1_ref, z2_ref, acc_ref):
    i = pl.program_id(0)
    sim = jnp.dot(z1_ref[...], z2_ref[...].T,
                  preferred_element_type=jnp.float32) * 2.0
    mx = jnp.max(sim, axis=1, keepdims=True)
    lse = jnp.log(jnp.sum(jnp.exp(sim - mx), axis=1)) + mx[:, 0]
    rows = i * _CL_BLK + lax.broadcasted_iota(jnp.int32, (_CL_BLK, 1), 0)
    cols = lax.broadcasted_iota(jnp.int32, (_CL_BLK, sim.shape[1]), 1)
    diag = jnp.sum(jnp.where(cols == rows, sim, 0.0), axis=1)
    part = jnp.sum(lse - diag)

    @pl.when(i == 0)
    def _():
        acc_ref[0, 0] = 0.0

    acc_ref[0, 0] += part


def _cl_loss(z1, z2):
    n = z1.shape[0]
    acc = pl.pallas_call(
        _cl_body,
        grid=(n // _CL_BLK,),
        in_specs=[pl.BlockSpec((_CL_BLK, HID), lambda i: (i, 0)),
                  pl.BlockSpec((n, HID), lambda i: (0, 0))],
        out_specs=pl.BlockSpec((1, 1), lambda i: (0, 0), memory_space=pltpu.SMEM),
        out_shape=jax.ShapeDtypeStruct((1, 1), jnp.float32),
    )(z1, z2)
    return acc[0, 0] / n


def _pad_edges(src, dst, e_pad, dummy):
    e = src.shape[0]
    src_p = jnp.concatenate([src, jnp.zeros((e_pad - e,), jnp.int32)])
    dst_p = jnp.concatenate([dst, jnp.full((e_pad - e,), dummy, jnp.int32)])
    return src_p, dst_p


def _assemble(out_b, nd):
    return out_b.transpose(1, 0, 2).reshape(out_b.shape[1], HID)[:nd]


def kernel(x_user, x_item, x_taste, x_intention, x_image, nutrient,
           ei_taste_ing, ei_taste_item, ei_int_item, ei_img_item,
           ei_user_item, ei_item_user, params):
    p = params
    D = float(HID)

    nut = _bn(jax.nn.relu(nutrient @ p['Wnp'] + p['bnp']), p['nbn_g'], p['nbn_b'])
    z1 = _encoder(nut, p)
    z2 = _encoder(x_intention, p)
    cl_loss = _cl_loss(z1, z2)

    xb = {'user': _bn(x_user, p['bn_user_g'], p['bn_user_b']),
          'item': _bn(x_item, p['bn_item_g'], p['bn_item_b']),
          'intention': _bn(x_intention, p['bn_int_g'], p['bn_int_b']),
          'image': _bn(x_image, p['bn_img_g'], p['bn_img_b']),
          'taste': _bn(x_taste, p['bn_taste_g'], p['bn_taste_b'])}

    NT = xb['taste'].shape[0]
    NDP_T = 32768
    E_LG = ei_taste_ing.shape[1]
    EP_LG = 483328
    lsrc, ldst = _pad_edges(ei_taste_ing[0], ei_taste_ing[1], EP_LG, NT)
    deg_part = _mk_deg(EP_LG, NDP_T)(ldst)
    deg = deg_part.sum(axis=0)[:NT]
    dinv = jnp.where(deg > 0, 1.0 / jnp.sqrt(jnp.maximum(deg, 1e-12)), 0.0)
    dinv_pad = jnp.concatenate([dinv, jnp.zeros((NDP_T - NT,), jnp.float32)])
    xt = xb['taste']
    xt_pad = jnp.concatenate(
        [xt, jnp.zeros((NDP_T - NT, HID), jnp.float32)], axis=0)
    tv = [xt_pad[:, 32 * j:32 * (j + 1)] for j in range(4)]
    lg_b = _mk_phase_b(EP_LG, NDP_T, lg=True, DT_pad=NDP_T)(
        lsrc, ldst, dinv_pad, *tv)
    taste_out = _assemble(lg_b, NT)

    xh = dict(xb)
    xh['taste'] = taste_out
    k_rel, v_rel = {}, {}
    for et in EDGE_TYPES:
        s_t, _, _ = et
        name = _ename(et)
        cc = p['prel_' + name] / jnp.sqrt(D)
        kk = (xh[s_t] @ p['Wk_' + s_t] + p['bk_' + s_t]) @ p['Ak_' + name]
        k_rel[name] = kk * cc
        v_rel[name] = (xh[s_t] @ p['Wv_' + s_t] + p['bv_' + s_t]) @ p['Av_' + name]
    q_item = xh['item'] @ p['Wq_item'] + p['bq_item']
    q_user = xh['user'] @ p['Wq_user'] + p['bq_user']

    rel_item = ['taste__associated_with__item', 'intention__associated_with__item',
                'image__associated_with__item', 'user__buys__item']
    ei_item = [ei_taste_item, ei_int_item, ei_img_item, ei_user_item]
    K_item = jnp.concatenate([k_rel[r] for r in rel_item], axis=0)
    V_item = jnp.concatenate([v_rel[r] for r in rel_item], axis=0)
    offs = [0]
    for r in rel_item[:-1]:
        offs.append(offs[-1] + k_rel[r].shape[0])
    src_item = jnp.concatenate(
        [e[0] + o for e, o in zip(ei_item, offs)])
    dst_item = jnp.concatenate([e[1] for e in ei_item])

    def _hgt_dst(src_g, dst_g, Ktab, Vtab, qtab, e_pad, nd, nd_pad, wa, ba,
                 skip, x_prev):
        src_p, dst_p = _pad_edges(src_g, dst_g, e_pad, nd)
        q_pad = jnp.concatenate(
            [qtab, jnp.zeros((nd_pad - nd, HID), jnp.float32)], axis=0)
        ex, den_part = _mk_phase_a(e_pad, nd_pad)(src_p, dst_p, Ktab, q_pad)
        vv = [Vtab[:, 32 * j:32 * (j + 1)] for j in range(4)]
        agg_b = _mk_phase_b(e_pad, nd_pad, lg=False)(src_p, dst_p, ex, *vv)
        agg = _assemble(agg_b, nd)
        den = den_part.sum(axis=0)[:nd]
        aggn = agg / jnp.maximum(den, 1e-16)[:, None]
        a = jax.nn.gelu(aggn, approximate=False) @ wa + ba
        beta = jax.nn.sigmoid(skip)
        return beta * a + (1.0 - beta) * x_prev

    item_out = _hgt_dst(src_item, dst_item, K_item, V_item, q_item,
                        1720320, 50000, 53248, p['Wa_item'], p['ba_item'],
                        p['skip_item'], xh['item'])
    user_out = _hgt_dst(ei_item_user[0], ei_item_user[1],
                        k_rel['item__bought_by__user'],
                        v_rel['item__bought_by__user'], q_user,
                        647168, 20000, 20480, p['Wa_user'], p['ba_user'],
                        p['skip_user'], xh['user'])

    return (user_out, item_out, taste_out, xb['intention'], xb['image'], cl_loss)


# speedup vs baseline: 7.9711x; 1.2139x over previous
import functools

import jax
import jax.numpy as jnp
from jax import lax
from jax.experimental import pallas as pl
from jax.experimental.pallas import tpu as pltpu
from jax.experimental.pallas import tpu_sc as plsc

HID = 128
NC, NS, NW = 2, 16, 32
BA = 128
BB = 128

EDGE_TYPES = [('taste', 'associated_with', 'item'),
              ('intention', 'associated_with', 'item'),
              ('image', 'associated_with', 'item'),
              ('user', 'buys', 'item'),
              ('item', 'bought_by', 'user')]


def _ename(et):
    return et[0] + '__' + et[1] + '__' + et[2]


def _bn(x, g, b, eps=1e-5):
    m = jnp.mean(x, axis=0)
    v = jnp.var(x, axis=0)
    return (x - m) / jnp.sqrt(v + eps) * g + b


def _encoder(x, p):
    h = jax.nn.relu(x @ p['ce_W1'] + p['ce_b1'])
    z = h @ p['ce_W2'] + p['ce_b2']
    nrm = jnp.sqrt(jnp.sum(z * z, axis=1, keepdims=True))
    return z / jnp.maximum(nrm, 1e-12)


def _mesh():
    return plsc.VectorSubcoreMesh(core_axis_name="c", subcore_axis_name="s",
                                  num_cores=NC, num_subcores=NS)


_SC_PARAMS = pltpu.CompilerParams(needs_layout_passes=False,
                                  use_tc_tiling_on_sc=False)


def _mk_phase_a(E_pad, ND_pad):
    Esh = E_pad // NW
    NB = Esh // BA
    assert Esh % BA == 0 and NB % 2 == 0 and ND_pad % 16 == 0

    @functools.partial(
        pl.kernel,
        out_type=(jax.ShapeDtypeStruct((E_pad,), jnp.float32),
                  jax.ShapeDtypeStruct((NW, ND_pad), jnp.float32)),
        mesh=_mesh(),
        compiler_params=_SC_PARAMS,
        scratch_types=[pltpu.VMEM((1, BA), jnp.int32),
                       pltpu.VMEM((1, BA), jnp.int32),
                       pltpu.VMEM((1, BA), jnp.int32),
                       pltpu.VMEM((1, BA), jnp.int32),
                       pltpu.VMEM((BA,), jnp.float32),
                       pltpu.VMEM((BA, HID), jnp.float32),
                       pltpu.VMEM((BA, HID), jnp.float32),
                       pltpu.VMEM((BA, HID), jnp.float32),
                       pltpu.VMEM((BA, HID), jnp.float32),
                       pltpu.VMEM((16, 137), jnp.float32),
                       pltpu.VMEM((ND_pad,), jnp.float32),
                       pltpu.SemaphoreType.DMA,
                       pltpu.SemaphoreType.DMA,
                       pltpu.SemaphoreType.DMA,
                       pltpu.SemaphoreType.DMA],
    )
    def kern(srcg, dst, ktab, qtab, ex_out, den_out,
             src_i0, dst_i0, src_i1, dst_i1, ex_b,
             krows0, qrows0, krows1, qrows1, acc_t, den_v,
             semk0, semq0, semk1, semq1):
        c = lax.axis_index("c")
        s = lax.axis_index("s")
        wid = s * NC + c
        base_t = wid * Esh
        iota16 = lax.iota(jnp.int32, 16)
        zeros16 = jnp.zeros((16,), jnp.float32)
        src_i = (src_i0, src_i1)
        dst_i = (dst_i0, dst_i1)
        krows = (krows0, krows1)
        qrows = (qrows0, qrows1)
        semk = (semk0, semk1)
        semq = (semq0, semq1)

        def zden(i, carry):
            plsc.store_scatter(den_v, [i * 16 + iota16], zeros16)
            return carry
        lax.fori_loop(0, ND_pad // 16, zden, 0)

        def fire(bi, b):
            base = base_t + bi * BA
            pltpu.sync_copy(srcg.at[pl.ds(base, BA)], src_i[b].at[0])
            pltpu.sync_copy(dst.at[pl.ds(base, BA)], dst_i[b].at[0])
            pltpu.async_copy(ktab.at[src_i[b].at[0]], krows[b], semk[b])
            pltpu.async_copy(qtab.at[dst_i[b].at[0]], qrows[b], semq[b])

        def wait(b):
            pltpu.make_async_copy(ktab.at[pl.ds(0, BA)], krows[b], semk[b]).wait()
            pltpu.make_async_copy(qtab.at[pl.ds(0, BA)], qrows[b], semq[b]).wait()

        def compute(bi, b):
            kr, qr = krows[b], qrows[b]
            for g in range(BA // 16):

                def edot(i, carry):
                    ev = jnp.full((16,), g * 16 + i, jnp.int32)
                    acc = zeros16
                    for cc in range(8):
                        fvec = 16 * cc + iota16
                        kv = plsc.load_gather(kr, [ev, fvec])
                        qv = plsc.load_gather(qr, [ev, fvec])
                        acc = acc + kv * qv
                    plsc.store_scatter(
                        acc_t, [iota16, jnp.full((16,), i, jnp.int32)], acc)
                    return carry
                lax.fori_loop(0, 16, edot, 0)
                sum16 = acc_t[0, pl.ds(0, 16)]
                for f in range(1, 16):
                    sum16 = sum16 + acc_t[f, pl.ds(0, 16)]
                ex_vec = jnp.exp(sum16)
                ex_b[pl.ds(g * 16, 16)] = ex_vec
                dvec = dst_i[b][0, pl.ds(g * 16, 16)]
                plsc.addupdate_scatter(den_v, [dvec], ex_vec)
            pltpu.sync_copy(ex_b, ex_out.at[pl.ds(base_t + bi * BA, BA)])

        fire(0, 0)

        def blk2(i, carry):
            wait(0)

            fire(2 * i + 1, 1)
            compute(2 * i, 0)
            wait(1)

            @pl.when(2 * i + 2 < NB)
            def _():
                fire(2 * i + 2, 0)
            compute(2 * i + 1, 1)
            return carry
        lax.fori_loop(0, NB // 2, blk2, 0)
        pltpu.sync_copy(den_v, den_out.at[wid])

    return kern


def _mk_phase_b(E_pad, ND_pad, lg, DT_pad=0):
    Esh = E_pad // NS
    NB = Esh // BB
    r_t = ND_pad // NS
    assert Esh % BB == 0 and ND_pad % (NS * 64) == 0

    scratch = [pltpu.VMEM((1, BB), jnp.int32),
               pltpu.VMEM((1, BB), jnp.int32),
               pltpu.VMEM((1, BB), jnp.int32),
               pltpu.VMEM((1, BB), jnp.int32),
               pltpu.VMEM((1, BB), jnp.float32),
               pltpu.VMEM((1, BB), jnp.float32),
               pltpu.VMEM((BB, 32), jnp.float32),
               pltpu.VMEM((BB, 32), jnp.float32),
               pltpu.VMEM((64, 32), jnp.float32),
               pltpu.VMEM_SHARED((ND_pad, 32), jnp.float32),
               pltpu.SemaphoreType.DMA,
               pltpu.SemaphoreType.DMA,
               pltpu.SemaphoreType.DMA,
               pltpu.SemaphoreType.DMA]
    if lg:
        scratch.append(pltpu.VMEM((DT_pad,), jnp.float32))

    @functools.partial(
        pl.kernel,
        out_type=jax.ShapeDtypeStruct((4, ND_pad, 32), jnp.float32),
        mesh=_mesh(),
        compiler_params=_SC_PARAMS,
        scratch_types=scratch,
    )
    def kern(srcg, dst, warr, v0, v1, v2, v3, out, *scr):
        if lg:
            (src_i0, dst_i0, src_i1, dst_i1, w_b0, w_b1, vrows0, vrows1,
             zbuf, agg, semg0, semg1, sems0, sems1, dinv_v) = scr
        else:
            (src_i0, dst_i0, src_i1, dst_i1, w_b0, w_b1, vrows0, vrows1,
             zbuf, agg, semg0, semg1, sems0, sems1) = scr
        src_i = (src_i0, src_i1)
        dst_i = (dst_i0, dst_i1)
        w_b = (w_b0, w_b1)
        vrows = (vrows0, vrows1)
        semg = (semg0, semg1)
        sems = (sems0, sems1)
        vs = (v0, v1, v2, v3)
        c = lax.axis_index("c")
        s = lax.axis_index("s")
        iota16 = lax.iota(jnp.int32, 16)
        z16i = jnp.zeros((16,), jnp.int32)
        zeros16 = jnp.zeros((16,), jnp.float32)
        for r in range(64):
            zbuf[r, pl.ds(0, 16)] = zeros16
            zbuf[r, pl.ds(16, 16)] = zeros16
        if lg:
            pltpu.sync_copy(warr, dinv_v)

        for chunk in range(4):
            @pl.when(c == chunk // 2)
            def _(chunk=chunk):
                def zrow(i, carry):
                    pltpu.sync_copy(zbuf, agg.at[pl.ds(s * r_t + i * 64, 64)])
                    return carry
                lax.fori_loop(0, r_t // 64, zrow, 0)
                plsc.subcore_barrier()

                vtab = vs[chunk]

                def fire_in(bi, b):
                    base = s * Esh + bi * BB
                    pltpu.sync_copy(srcg.at[pl.ds(base, BB)], src_i[b].at[0])
                    pltpu.sync_copy(dst.at[pl.ds(base, BB)], dst_i[b].at[0])
                    if not lg:
                        pltpu.sync_copy(warr.at[pl.ds(base, BB)], w_b[b].at[0])
                    pltpu.async_copy(vtab.at[src_i[b].at[0]], vrows[b], semg[b])

                def wait_g(b):
                    pltpu.make_async_copy(
                        vtab.at[pl.ds(0, BB)], vrows[b], semg[b]).wait()

                def fire_sc(b):
                    pltpu.async_copy(vrows[b], agg.at[dst_i[b].at[0]],
                                     sems[b], add=True)

                def wait_sc(b):
                    pltpu.make_async_copy(
                        vrows[b], agg.at[pl.ds(0, BB)], sems[b]).wait()

                def scale(b):
                    if lg:
                        for g in range(BB // 16):
                            svec = src_i[b][0, pl.ds(g * 16, 16)]
                            dvec = dst_i[b][0, pl.ds(g * 16, 16)]
                            wv = (plsc.load_gather(dinv_v, [svec]) *
                                  plsc.load_gather(dinv_v, [dvec]))
                            w_b[b][0, pl.ds(g * 16, 16)] = wv

                    def sca(e, carry):
                        ev = jnp.full((16,), e, jnp.int32)
                        w16 = plsc.load_gather(w_b[b], [z16i, ev])
                        lo = plsc.load_gather(vrows[b], [ev, iota16])
                        hi = plsc.load_gather(vrows[b], [ev, iota16 + 16])
                        plsc.store_scatter(vrows[b], [ev, iota16], lo * w16)
                        plsc.store_scatter(vrows[b], [ev, iota16 + 16],
                                           hi * w16)
                        return carry
                    lax.fori_loop(0, BB, sca, 0)

                fire_in(0, 0)

                def blk2(i, carry):
                    wait_g(0)

                    @pl.when(i > 0)
                    def _():
                        wait_sc(1)
                    fire_in(2 * i + 1, 1)
                    scale(0)
                    fire_sc(0)
                    wait_g(1)
                    wait_sc(0)

                    @pl.when(2 * i + 2 < NB)
                    def _():
                        fire_in(2 * i + 2, 0)
                    scale(1)
                    fire_sc(1)
                    return carry
                lax.fori_loop(0, NB // 2, blk2, 0)
                wait_sc(1)
                plsc.subcore_barrier()
                pltpu.sync_copy(agg.at[pl.ds(s * r_t, r_t)],
                                out.at[chunk, pl.ds(s * r_t, r_t)])
                plsc.subcore_barrier()

    return kern


def _mk_deg(E_pad, ND_pad):
    Esh = E_pad // NW
    NB = Esh // BA

    @functools.partial(
        pl.kernel,
        out_type=jax.ShapeDtypeStruct((NW, ND_pad), jnp.float32),
        mesh=_mesh(),
        compiler_params=_SC_PARAMS,
        scratch_types=[pltpu.VMEM((1, BA), jnp.int32),
                       pltpu.VMEM((ND_pad,), jnp.float32)],
    )
    def kern(dst, deg_out, dst_i, deg_v):
        c = lax.axis_index("c")
        s = lax.axis_index("s")
        wid = s * NC + c
        base_t = wid * Esh
        iota16 = lax.iota(jnp.int32, 16)
        zeros16 = jnp.zeros((16,), jnp.float32)
        ones16 = jnp.ones((16,), jnp.float32)

        def zden(i, carry):
            plsc.store_scatter(deg_v, [i * 16 + iota16], zeros16)
            return carry
        lax.fori_loop(0, ND_pad // 16, zden, 0)

        def blk(bi, carry):
            pltpu.sync_copy(dst.at[pl.ds(base_t + bi * BA, BA)], dst_i.at[0])
            for g in range(BA // 16):
                dvec = dst_i[0, pl.ds(g * 16, 16)]
                plsc.addupdate_scatter(deg_v, [dvec], ones16)
            return carry
        lax.fori_loop(0, NB, blk, 0)
        pltpu.sync_copy(deg_v, deg_out.at[wid])

    return kern


_CL_BLK = 512


def _cl_body(z1_ref, z2_ref, acc_ref):
    i = pl.program_id(0)
    sim = jnp.dot(z1_ref[...], z2_ref[...].T,
                  preferred_element_type=jnp.float32) * 2.0
    mx = jnp.max(sim, axis=1, keepdims=True)
    lse = jnp.log(jnp.sum(jnp.exp(sim - mx), axis=1)) + mx[:, 0]
    rows = i * _CL_BLK + lax.broadcasted_iota(jnp.int32, (_CL_BLK, 1), 0)
    cols = lax.broadcasted_iota(jnp.int32, (_CL_BLK, sim.shape[1]), 1)
    diag = jnp.sum(jnp.where(cols == rows, sim, 0.0), axis=1)
    part = jnp.sum(lse - diag)

    @pl.when(i == 0)
    def _():
        acc_ref[0, 0] = 0.0

    acc_ref[0, 0] += part


def _cl_loss(z1, z2):
    n = z1.shape[0]
    acc = pl.pallas_call(
        _cl_body,
        grid=(n // _CL_BLK,),
        in_specs=[pl.BlockSpec((_CL_BLK, HID), lambda i: (i, 0)),
                  pl.BlockSpec((n, HID), lambda i: (0, 0))],
        out_specs=pl.BlockSpec((1, 1), lambda i: (0, 0), memory_space=pltpu.SMEM),
        out_shape=jax.ShapeDtypeStruct((1, 1), jnp.float32),
    )(z1, z2)
    return acc[0, 0] / n


def _pad_edges(src, dst, e_pad, dummy):
    e = src.shape[0]
    src_p = jnp.concatenate([src, jnp.zeros((e_pad - e,), jnp.int32)])
    dst_p = jnp.concatenate([dst, jnp.full((e_pad - e,), dummy, jnp.int32)])
    return src_p, dst_p


def _assemble(out_b, nd):
    return out_b.transpose(1, 0, 2).reshape(out_b.shape[1], HID)[:nd]


def kernel(x_user, x_item, x_taste, x_intention, x_image, nutrient,
           ei_taste_ing, ei_taste_item, ei_int_item, ei_img_item,
           ei_user_item, ei_item_user, params):
    p = params
    D = float(HID)

    nut = _bn(jax.nn.relu(nutrient @ p['Wnp'] + p['bnp']), p['nbn_g'], p['nbn_b'])
    z1 = _encoder(nut, p)
    z2 = _encoder(x_intention, p)
    cl_loss = _cl_loss(z1, z2)

    xb = {'user': _bn(x_user, p['bn_user_g'], p['bn_user_b']),
          'item': _bn(x_item, p['bn_item_g'], p['bn_item_b']),
          'intention': _bn(x_intention, p['bn_int_g'], p['bn_int_b']),
          'image': _bn(x_image, p['bn_img_g'], p['bn_img_b']),
          'taste': _bn(x_taste, p['bn_taste_g'], p['bn_taste_b'])}

    NT = xb['taste'].shape[0]
    NDP_T = 32768
    E_LG = ei_taste_ing.shape[1]
    EP_LG = 483328
    lsrc, ldst = _pad_edges(ei_taste_ing[0], ei_taste_ing[1], EP_LG, NT)
    deg_part = _mk_deg(EP_LG, NDP_T)(ldst)
    deg = deg_part.sum(axis=0)[:NT]
    dinv = jnp.where(deg > 0, 1.0 / jnp.sqrt(jnp.maximum(deg, 1e-12)), 0.0)
    dinv_pad = jnp.concatenate([dinv, jnp.zeros((NDP_T - NT,), jnp.float32)])
    xt = xb['taste']
    xt_pad = jnp.concatenate(
        [xt, jnp.zeros((NDP_T - NT, HID), jnp.float32)], axis=0)
    tv = [xt_pad[:, 32 * j:32 * (j + 1)] for j in range(4)]
    lg_b = _mk_phase_b(EP_LG, NDP_T, lg=True, DT_pad=NDP_T)(
        lsrc, ldst, dinv_pad, *tv)
    taste_out = _assemble(lg_b, NT)

    xh = dict(xb)
    xh['taste'] = taste_out
    k_rel, v_rel = {}, {}
    for et in EDGE_TYPES:
        s_t, _, _ = et
        name = _ename(et)
        cc = p['prel_' + name] / jnp.sqrt(D)
        kk = (xh[s_t] @ p['Wk_' + s_t] + p['bk_' + s_t]) @ p['Ak_' + name]
        k_rel[name] = kk * cc
        v_rel[name] = (xh[s_t] @ p['Wv_' + s_t] + p['bv_' + s_t]) @ p['Av_' + name]
    q_item = xh['item'] @ p['Wq_item'] + p['bq_item']
    q_user = xh['user'] @ p['Wq_user'] + p['bq_user']

    rel_item = ['taste__associated_with__item', 'intention__associated_with__item',
                'image__associated_with__item', 'user__buys__item']
    ei_item = [ei_taste_item, ei_int_item, ei_img_item, ei_user_item]
    K_item = jnp.concatenate([k_rel[r] for r in rel_item], axis=0)
    V_item = jnp.concatenate([v_rel[r] for r in rel_item], axis=0)
    offs = [0]
    for r in rel_item[:-1]:
        offs.append(offs[-1] + k_rel[r].shape[0])
    src_item = jnp.concatenate(
        [e[0] + o for e, o in zip(ei_item, offs)])
    dst_item = jnp.concatenate([e[1] for e in ei_item])

    def _hgt_dst(src_g, dst_g, Ktab, Vtab, qtab, e_pad, nd, nd_pad, wa, ba,
                 skip, x_prev):
        src_p, dst_p = _pad_edges(src_g, dst_g, e_pad, nd)
        q_pad = jnp.concatenate(
            [qtab, jnp.zeros((nd_pad - nd, HID), jnp.float32)], axis=0)
        ex, den_part = _mk_phase_a(e_pad, nd_pad)(src_p, dst_p, Ktab, q_pad)
        vv = [Vtab[:, 32 * j:32 * (j + 1)] for j in range(4)]
        agg_b = _mk_phase_b(e_pad, nd_pad, lg=False)(src_p, dst_p, ex, *vv)
        agg = _assemble(agg_b, nd)
        den = den_part.sum(axis=0)[:nd]
        aggn = agg / jnp.maximum(den, 1e-16)[:, None]
        a = jax.nn.gelu(aggn, approximate=False) @ wa + ba
        beta = jax.nn.sigmoid(skip)
        return beta * a + (1.0 - beta) * x_prev

    item_out = _hgt_dst(src_item, dst_item, K_item, V_item, q_item,
                        1720320, 50000, 53248, p['Wa_item'], p['ba_item'],
                        p['skip_item'], xh['item'])
    user_out = _hgt_dst(ei_item_user[0], ei_item_user[1],
                        k_rel['item__bought_by__user'],
                        v_rel['item__bought_by__user'], q_user,
                        647168, 20000, 20480, p['Wa_user'], p['ba_user'],
                        p['skip_user'], xh['user'])

    return (user_out, item_out, taste_out, xb['intention'], xb['image'], cl_loss)
